# Initial kernel scaffold; baseline (speedup 1.0000x reference)
import jax
import jax.numpy as jnp
from jax.experimental import pallas as pl


def kernel(x, edge_index, edge_attr, u, batch, W, b):
    raise NotImplementedError("write your pallas kernel here")


# speedup vs baseline: 14.1533x; 14.1533x over previous
import functools

import jax
import jax.numpy as jnp
from jax import lax
from jax.experimental import pallas as pl
from jax.experimental.pallas import tpu as pltpu
from jax.experimental.pallas import tpu_sc as plsc

_NN = 100000
_NE = 1600000
_NG = 256
_DN = 128
_DE = 32
_DO = 64

_NW = 32
_EPW = _NE // _NW
_C = 400
_NCH = _EPW // _C
_GRP = _C // 16

_sc_mesh = plsc.VectorSubcoreMesh(core_axis_name="c", subcore_axis_name="s")


@functools.partial(
    pl.kernel,
    mesh=_sc_mesh,
    compiler_params=pltpu.CompilerParams(needs_layout_passes=False),
    out_type=jax.ShapeDtypeStruct((_NW, _NG * _DE), jnp.float32),
    scratch_types=[
        pltpu.VMEM((_NN,), jnp.int32),
        pltpu.VMEM((_C,), jnp.int32),
        pltpu.VMEM((_C * _DE,), jnp.float32),
        pltpu.VMEM((_NG * _DE,), jnp.float32),
    ],
)
def _edge_agg(col_hbm, ea_hbm, batch_hbm, out_hbm, batch_v, col_v, ea_v,
              acc_v):
    wid = lax.axis_index("s") * 2 + lax.axis_index("c")
    pltpu.sync_copy(batch_hbm, batch_v)

    zeros = jnp.zeros((16,), jnp.float32)

    def zero_body(k, carry):
        acc_v[pl.ds(k * 16, 16)] = zeros
        return carry

    lax.fori_loop(0, _NG * _DE // 16, zero_body, 0)

    iota = lax.iota(jnp.int32, 16)

    def chunk_body(i, carry):
        base = wid * _EPW + i * _C
        pltpu.sync_copy(col_hbm.at[pl.ds(base, _C)], col_v)
        pltpu.sync_copy(ea_hbm.at[pl.ds(base * _DE, _C * _DE)], ea_v)

        def grp_body(g, c2):
            cv = col_v[pl.ds(g * 16, 16)]
            gv = plsc.load_gather(batch_v, [cv]) * _DE
            for j in range(16):
                row = gv[j]
                e = g * 16 + j
                v0 = ea_v[pl.ds(e * _DE, 16)]
                v1 = ea_v[pl.ds(e * _DE + 16, 16)]
                plsc.addupdate_scatter(acc_v, [row + iota], v0)
                plsc.addupdate_scatter(acc_v, [row + (iota + 16)], v1)
            return c2

        lax.fori_loop(0, _GRP, grp_body, 0)
        return carry

    lax.fori_loop(0, _NCH, chunk_body, 0)
    pltpu.sync_copy(acc_v, out_hbm.at[wid])


_R = 2000
_NS = _NN // _R


def _node_body(b_ref, x_ref, o_ref, acc_ref):
    s = pl.program_id(0)

    @pl.when(s == 0)
    def _():
        acc_ref[...] = jnp.zeros_like(acc_ref)

    bt = b_ref[0, 0, :]
    onehot = (lax.broadcasted_iota(jnp.int32, (_NG, _R), 0)
              == bt[None, :]).astype(jnp.float32)
    acc_ref[...] += jnp.dot(onehot, x_ref[...],
                            preferred_element_type=jnp.float32)

    @pl.when(s == _NS - 1)
    def _():
        o_ref[...] = acc_ref[...]


_node_call = pl.pallas_call(
    _node_body,
    grid=(_NS,),
    in_specs=[
        pl.BlockSpec((1, 1, _R), lambda i: (i, 0, 0)),
        pl.BlockSpec((_R, _DN), lambda i: (i, 0)),
    ],
    out_specs=pl.BlockSpec((_NG, _DN), lambda i: (0, 0)),
    out_shape=jax.ShapeDtypeStruct((_NG, _DN), jnp.float32),
    scratch_shapes=[pltpu.VMEM((_NG, _DN), jnp.float32)],
)


def _comb_body(nag_ref, ep_ref, w_ref, b_ref, o_ref):
    eag = jnp.sum(ep_ref[...], axis=0)
    out = jnp.dot(nag_ref[...], w_ref[0:_DN, :],
                  preferred_element_type=jnp.float32)
    out = out + jnp.dot(eag, w_ref[_DN:_DN + _DE, :],
                        preferred_element_type=jnp.float32)
    o_ref[...] = out + b_ref[...]


_comb_call = pl.pallas_call(
    _comb_body,
    out_shape=jax.ShapeDtypeStruct((_NG, _DO), jnp.float32),
)


@jax.jit
def kernel(x, edge_index, edge_attr, u, batch, W, b):
    col = edge_index[1]
    ep = _edge_agg(col, edge_attr.reshape(-1), batch)
    nag = _node_call(batch.reshape(_NS, 1, _R), x)
    return _comb_call(nag, ep.reshape(_NW, _NG, _DE), W,
                      b.reshape(1, _DO))


# speedup vs baseline: 14.1606x; 1.0005x over previous
import functools

import jax
import jax.numpy as jnp
from jax import lax
from jax.experimental import pallas as pl
from jax.experimental.pallas import tpu as pltpu
from jax.experimental.pallas import tpu_sc as plsc

_NN = 100000
_NE = 1600000
_NG = 256
_DN = 128
_DE = 32
_DO = 64

_NW = 32
_EPW = _NE // _NW
_C = 400
_NCH = _EPW // _C
_GRP = _C // 16

_sc_mesh = plsc.VectorSubcoreMesh(core_axis_name="c", subcore_axis_name="s")


@functools.partial(
    pl.kernel,
    mesh=_sc_mesh,
    compiler_params=pltpu.CompilerParams(needs_layout_passes=False,
                                         use_tc_tiling_on_sc=False),
    out_type=jax.ShapeDtypeStruct((_NW, _NG * _DE), jnp.float32),
    scratch_types=[
        pltpu.VMEM((_NN,), jnp.int32),
        pltpu.VMEM((_C,), jnp.int32),
        pltpu.VMEM((_C, _DE), jnp.float32),
        pltpu.VMEM((_NG * _DE,), jnp.float32),
    ],
)
def _edge_agg(col_hbm, ea_hbm, batch_hbm, out_hbm, batch_v, col_v, ea_v,
              acc_v):
    wid = lax.axis_index("s") * 2 + lax.axis_index("c")
    pltpu.sync_copy(batch_hbm, batch_v)

    zeros = jnp.zeros((16,), jnp.float32)

    def zero_body(k, carry):
        acc_v[pl.ds(k * 16, 16)] = zeros
        return carry

    lax.fori_loop(0, _NG * _DE // 16, zero_body, 0)

    iota = lax.iota(jnp.int32, 16)

    def chunk_body(i, carry):
        base = wid * _EPW + i * _C
        pltpu.sync_copy(col_hbm.at[pl.ds(base, _C)], col_v)
        pltpu.sync_copy(ea_hbm.at[pl.ds(base, _C), :], ea_v)

        def grp_body(g, c2):
            cv = col_v[pl.ds(g * 16, 16)]
            gv = plsc.load_gather(batch_v, [cv]) * _DE
            for j in range(16):
                row = gv[j]
                e = g * 16 + j
                v0 = ea_v[e, pl.ds(0, 16)]
                v1 = ea_v[e, pl.ds(16, 16)]
                plsc.addupdate_scatter(acc_v, [row + iota], v0)
                plsc.addupdate_scatter(acc_v, [row + (iota + 16)], v1)
            return c2

        lax.fori_loop(0, _GRP, grp_body, 0)
        return carry

    lax.fori_loop(0, _NCH, chunk_body, 0)
    pltpu.sync_copy(acc_v, out_hbm.at[wid])


_R = 2000
_NS = _NN // _R


def _node_body(b_ref, x_ref, o_ref, acc_ref):
    s = pl.program_id(0)

    @pl.when(s == 0)
    def _():
        acc_ref[...] = jnp.zeros_like(acc_ref)

    bt = b_ref[0, 0, :]
    onehot = (lax.broadcasted_iota(jnp.int32, (_NG, _R), 0)
              == bt[None, :]).astype(jnp.float32)
    acc_ref[...] += jnp.dot(onehot, x_ref[...],
                            preferred_element_type=jnp.float32)

    @pl.when(s == _NS - 1)
    def _():
        o_ref[...] = acc_ref[...]


_node_call = pl.pallas_call(
    _node_body,
    grid=(_NS,),
    in_specs=[
        pl.BlockSpec((1, 1, _R), lambda i: (i, 0, 0)),
        pl.BlockSpec((_R, _DN), lambda i: (i, 0)),
    ],
    out_specs=pl.BlockSpec((_NG, _DN), lambda i: (0, 0)),
    out_shape=jax.ShapeDtypeStruct((_NG, _DN), jnp.float32),
    scratch_shapes=[pltpu.VMEM((_NG, _DN), jnp.float32)],
)


def _comb_body(nag_ref, ep_ref, w_ref, b_ref, o_ref):
    eag = jnp.sum(ep_ref[...], axis=0)
    out = jnp.dot(nag_ref[...], w_ref[0:_DN, :],
                  preferred_element_type=jnp.float32)
    out = out + jnp.dot(eag, w_ref[_DN:_DN + _DE, :],
                        preferred_element_type=jnp.float32)
    o_ref[...] = out + b_ref[...]


_comb_call = pl.pallas_call(
    _comb_body,
    out_shape=jax.ShapeDtypeStruct((_NG, _DO), jnp.float32),
)


@jax.jit
def kernel(x, edge_index, edge_attr, u, batch, W, b):
    col = edge_index[1]
    ep = _edge_agg(col, edge_attr, batch)
    nag = _node_call(batch.reshape(_NS, 1, _R), x)
    return _comb_call(nag, ep.reshape(_NW, _NG, _DE), W,
                      b.reshape(1, _DO))
